# baseline (speedup 1.0000x reference)
import functools

import jax
import jax.numpy as jnp
from jax import lax
from jax.experimental import pallas as pl
from jax.experimental.pallas import tpu as pltpu
from jax.experimental.pallas import tpu_sc as plsc

N = 10000
E = 320000
D = 128
NC, NS = 2, 16
NW = NC * NS
CH = 125
NCHUNKS = E // CH
CPW = NCHUNKS // NW
NP = 10240
ROWS_PER_TILE = NP // NS
ZROWS = 40
WROWS = 128
G = 16

_mesh = plsc.VectorSubcoreMesh(
    core_axis_name="c", subcore_axis_name="s", num_cores=NC, num_subcores=NS)


def _worker_id():
    return lax.axis_index("s") * NC + lax.axis_index("c")


def _zero_acc(zbuf, acc, width):
    s = lax.axis_index("s")

    def fill_zero(i, _):
        for j in range(width // 16):
            zbuf[i, pl.ds(j * 16, 16)] = jnp.zeros((16,), jnp.float32)
        return 0

    lax.fori_loop(0, ZROWS, fill_zero, 0)

    def zchunk(j, _):
        pltpu.sync_copy(zbuf, acc.at[pl.ds(s * ROWS_PER_TILE + j * ZROWS, ZROWS)])
        return 0

    lax.fori_loop(0, ROWS_PER_TILE // ZROWS, zchunk, 0)


def _write_out(acc, out, core):
    s = lax.axis_index("s")

    def wchunk(j, _):
        r0 = s * ROWS_PER_TILE + j * WROWS
        pltpu.sync_copy(acc.at[pl.ds(r0, WROWS)], out.at[core, pl.ds(r0, WROWS)])
        return 0

    lax.fori_loop(0, ROWS_PER_TILE // WROWS, wchunk, 0)


_DEG_WIN = 8


def _deg_body(e3, degp, didxs, ones_v, zbuf, acc, sem):
    core = lax.axis_index("c")
    w = _worker_id()

    def fill_ones(i, _):
        for j in range(D // 16):
            ones_v[i, pl.ds(j * 16, 16)] = jnp.ones((16,), jnp.float32)
        return 0

    lax.fori_loop(0, CH, fill_ones, 0)
    _zero_acc(zbuf, acc, D)
    pltpu.sync_copy(e3.at[1, pl.ds(w * CPW, CPW)], didxs)
    plsc.subcore_barrier()

    def body(j, _):
        pltpu.async_copy(ones_v, acc.at[didxs.at[j]], sem, add=True)

        @pl.when(j >= _DEG_WIN)
        def _():
            pltpu.make_async_copy(ones_v, acc.at[didxs.at[0]], sem).wait()

        return 0

    lax.fori_loop(0, CPW, body, 0)

    def drain(j, _):
        pltpu.make_async_copy(ones_v, acc.at[didxs.at[0]], sem).wait()
        return 0

    lax.fori_loop(0, _DEG_WIN, drain, 0)
    plsc.subcore_barrier()
    _write_out(acc, degp, core)


_deg_call = functools.partial(
    pl.kernel,
    out_type=jax.ShapeDtypeStruct((NC, NP, D), jnp.float32),
    mesh=_mesh,
    scratch_types=[
        pltpu.VMEM((CPW, CH), jnp.int32),
        pltpu.VMEM((CH, D), jnp.float32),
        pltpu.VMEM((ZROWS, D), jnp.float32),
        pltpu.VMEM_SHARED((NP, D), jnp.float32),
        pltpu.SemaphoreType.DMA,
    ],
)(_deg_body)


def _prop_body(u, e3, sp, sidxs, didxs, rows0, rows1, zbuf, acc, g0, g1, isem):
    core = lax.axis_index("c")
    w = _worker_id()

    pltpu.async_copy(e3.at[0, pl.ds(w * CPW, G)], sidxs.at[0], isem)
    pltpu.async_copy(e3.at[1, pl.ds(w * CPW, G)], didxs.at[0], isem)
    _zero_acc(zbuf, acc, D)
    plsc.subcore_barrier()

    def group(gidx, _):
        par = gidx % 2
        pltpu.make_async_copy(e3.at[0, pl.ds(0, G)], sidxs.at[par], isem).wait()
        pltpu.make_async_copy(e3.at[1, pl.ds(0, G)], didxs.at[par], isem).wait()

        @pl.when(gidx + 1 < CPW // G)
        def _():
            nrow = w * CPW + (gidx + 1) * G
            pltpu.async_copy(e3.at[0, pl.ds(nrow, G)], sidxs.at[1 - par], isem)
            pltpu.async_copy(e3.at[1, pl.ds(nrow, G)], didxs.at[1 - par], isem)

        pltpu.async_copy(u.at[sidxs.at[par, 0]], rows0, g0)

        def body(i, _):
            j0 = 2 * i
            j1 = j0 + 1
            pltpu.async_copy(u.at[sidxs.at[par, j1]], rows1, g1)
            pltpu.make_async_copy(u.at[sidxs.at[par, j0]], rows0, g0).wait()
            pltpu.sync_copy(rows0, acc.at[didxs.at[par, j0]], add=True)

            @pl.when(i + 1 < G // 2)
            def _():
                pltpu.async_copy(u.at[sidxs.at[par, j0 + 2]], rows0, g0)

            pltpu.make_async_copy(u.at[sidxs.at[par, j1]], rows1, g1).wait()
            pltpu.sync_copy(rows1, acc.at[didxs.at[par, j1]], add=True)
            return 0

        lax.fori_loop(0, G // 2, body, 0)
        return 0

    lax.fori_loop(0, CPW // G, group, 0)
    plsc.subcore_barrier()
    _write_out(acc, sp, core)


_prop_call = functools.partial(
    pl.kernel,
    out_type=jax.ShapeDtypeStruct((NC, NP, D), jnp.float32),
    mesh=_mesh,
    scratch_types=[
        pltpu.VMEM((2, G, CH), jnp.int32),
        pltpu.VMEM((2, G, CH), jnp.int32),
        pltpu.VMEM((CH, D), jnp.float32),
        pltpu.VMEM((CH, D), jnp.float32),
        pltpu.VMEM((ZROWS, D), jnp.float32),
        pltpu.VMEM_SHARED((NP, D), jnp.float32),
        pltpu.SemaphoreType.DMA,
        pltpu.SemaphoreType.DMA,
        pltpu.SemaphoreType.DMA,
    ],
)(_prop_body)


BN = 5000
_GRID = N // BN


def _row_specs(n, shape=(BN, D)):
    return [pl.BlockSpec(shape, lambda i: (i, 0)) for _ in range(n)]


_PART_SPEC = pl.BlockSpec((NC, BN, D), lambda i: (0, i, 0))


def _norm_body(dp, x, normf, u0):
    d = dp[...]
    degc = d[0, :, 0:1] + d[1, :, 0:1]
    nrm = jnp.where(degc > 0, lax.rsqrt(jnp.maximum(degc, 1.0)), 0.0)
    nb = jnp.broadcast_to(nrm, (BN, D))
    normf[...] = nb
    u0[...] = x[...] * nb


def _norm_call(dp, x):
    return pl.pallas_call(
        _norm_body,
        grid=(_GRID,),
        in_specs=[_PART_SPEC] + _row_specs(1),
        out_specs=_row_specs(2),
        out_shape=[jax.ShapeDtypeStruct((N, D), jnp.float32)] * 2,
    )(dp, x)


def _mid_body(sp, nrm, h1, u1):
    spv = sp[...]
    s = (spv[0] + spv[1]) * nrm[...]
    h1[...] = s
    u1[...] = s * nrm[...]


def _mid_call(sp, nrm):
    return pl.pallas_call(
        _mid_body,
        grid=(_GRID,),
        in_specs=[_PART_SPEC] + _row_specs(1),
        out_specs=_row_specs(2),
        out_shape=[jax.ShapeDtypeStruct((N, D), jnp.float32)] * 2,
    )(sp, nrm)


def _out_body(h, h1, s2p, nrm, w_ref, b_ref, hout, uout):
    s2v = s2p[...]
    h2 = (s2v[0] + s2v[1]) * nrm[...]
    wm = w_ref[...]
    acc = jnp.dot(h[...], wm[0:D], preferred_element_type=jnp.float32,
                  precision=lax.Precision.HIGHEST)
    acc += jnp.dot(h1[...], wm[D:2 * D], preferred_element_type=jnp.float32,
                   precision=lax.Precision.HIGHEST)
    acc += jnp.dot(h2, wm[2 * D:3 * D], preferred_element_type=jnp.float32,
                   precision=lax.Precision.HIGHEST)
    o = jnp.maximum(acc + b_ref[...], 0.0)
    hout[...] = o
    uout[...] = o * nrm[...]


def _out_call(h, h1, s2p, nrm, wmat, bias):
    return pl.pallas_call(
        _out_body,
        grid=(_GRID,),
        in_specs=_row_specs(2) + [_PART_SPEC] + _row_specs(1) + [
            pl.BlockSpec((3 * D, D), lambda i: (0, 0)),
            pl.BlockSpec((1, D), lambda i: (0, 0)),
        ],
        out_specs=_row_specs(2),
        out_shape=[jax.ShapeDtypeStruct((N, D), jnp.float32)] * 2,
    )(h, h1, s2p, nrm, wmat, bias)


def kernel(x, edge_index, W1, b1, W2, b2):
    e3 = edge_index.reshape(2, NCHUNKS, CH)
    degp = _deg_call(e3)
    nrm, u0 = _norm_call(degp, x)

    s1 = _prop_call(u0, e3)
    h1, u1 = _mid_call(s1, nrm)
    s2 = _prop_call(u1, e3)
    hL1, uL1 = _out_call(x, h1, s2, nrm, W1, b1.reshape(1, D))

    s3 = _prop_call(uL1, e3)
    h21, u21 = _mid_call(s3, nrm)
    s4 = _prop_call(u21, e3)
    hL2, _ = _out_call(hL1, h21, s4, nrm, W2, b2.reshape(1, D))
    return hL2


# speedup vs baseline: 1.0355x; 1.0355x over previous
import functools

import jax
import jax.numpy as jnp
from jax import lax
from jax.experimental import pallas as pl
from jax.experimental.pallas import tpu as pltpu
from jax.experimental.pallas import tpu_sc as plsc

N = 10000
E = 320000
D = 128
NC, NS = 2, 16
NW = NC * NS
CH = 125
NCHUNKS = E // CH
CPW = NCHUNKS // NW
NP = 10240
ROWS_PER_TILE = NP // NS
ZROWS = 40
WROWS = 128
G = 16

_mesh = plsc.VectorSubcoreMesh(
    core_axis_name="c", subcore_axis_name="s", num_cores=NC, num_subcores=NS)


def _worker_id():
    return lax.axis_index("s") * NC + lax.axis_index("c")


def _zero_acc(zbuf, acc, width):
    s = lax.axis_index("s")

    def fill_zero(i, _):
        for j in range(width // 16):
            zbuf[i, pl.ds(j * 16, 16)] = jnp.zeros((16,), jnp.float32)
        return 0

    lax.fori_loop(0, ZROWS, fill_zero, 0)

    def zchunk(j, _):
        pltpu.sync_copy(zbuf, acc.at[pl.ds(s * ROWS_PER_TILE + j * ZROWS, ZROWS)])
        return 0

    lax.fori_loop(0, ROWS_PER_TILE // ZROWS, zchunk, 0)


def _write_out(acc, out, core):
    s = lax.axis_index("s")

    def wchunk(j, _):
        r0 = s * ROWS_PER_TILE + j * WROWS
        pltpu.sync_copy(acc.at[pl.ds(r0, WROWS)], out.at[core, pl.ds(r0, WROWS)])
        return 0

    lax.fori_loop(0, ROWS_PER_TILE // WROWS, wchunk, 0)


_DEG_WIN = 8


def _deg_body(e3, degp, didxs, ones_v, zbuf, acc, sem):
    core = lax.axis_index("c")
    w = _worker_id()

    def fill_ones(i, _):
        for j in range(D // 16):
            ones_v[i, pl.ds(j * 16, 16)] = jnp.ones((16,), jnp.float32)
        return 0

    lax.fori_loop(0, CH, fill_ones, 0)
    _zero_acc(zbuf, acc, D)
    pltpu.sync_copy(e3.at[1, pl.ds(w * CPW, CPW)], didxs)
    plsc.subcore_barrier()

    def body(j, _):
        pltpu.async_copy(ones_v, acc.at[didxs.at[j]], sem, add=True)

        @pl.when(j >= _DEG_WIN)
        def _():
            pltpu.make_async_copy(ones_v, acc.at[didxs.at[0]], sem).wait()

        return 0

    lax.fori_loop(0, CPW, body, 0)

    def drain(j, _):
        pltpu.make_async_copy(ones_v, acc.at[didxs.at[0]], sem).wait()
        return 0

    lax.fori_loop(0, _DEG_WIN, drain, 0)
    plsc.subcore_barrier()
    _write_out(acc, degp, core)


_deg_call = functools.partial(
    pl.kernel,
    out_type=jax.ShapeDtypeStruct((NC, NP, D), jnp.float32),
    mesh=_mesh,
    scratch_types=[
        pltpu.VMEM((CPW, CH), jnp.int32),
        pltpu.VMEM((CH, D), jnp.float32),
        pltpu.VMEM((ZROWS, D), jnp.float32),
        pltpu.VMEM_SHARED((NP, D), jnp.float32),
        pltpu.SemaphoreType.DMA,
    ],
)(_deg_body)


def _prop_body(u, e3, sp, sidxs, didxs, rows0, rows1, zbuf, acc, g0, g1, isem):
    core = lax.axis_index("c")
    w = _worker_id()

    pltpu.async_copy(e3.at[0, pl.ds(w * CPW, G)], sidxs.at[0], isem)
    pltpu.async_copy(e3.at[1, pl.ds(w * CPW, G)], didxs.at[0], isem)
    _zero_acc(zbuf, acc, D)
    plsc.subcore_barrier()

    def group(gidx, _):
        par = gidx % 2
        pltpu.make_async_copy(e3.at[0, pl.ds(0, G)], sidxs.at[par], isem).wait()
        pltpu.make_async_copy(e3.at[1, pl.ds(0, G)], didxs.at[par], isem).wait()

        @pl.when(gidx + 1 < CPW // G)
        def _():
            nrow = w * CPW + (gidx + 1) * G
            pltpu.async_copy(e3.at[0, pl.ds(nrow, G)], sidxs.at[1 - par], isem)
            pltpu.async_copy(e3.at[1, pl.ds(nrow, G)], didxs.at[1 - par], isem)

        pltpu.async_copy(u.at[sidxs.at[par, 0]], rows0, g0)

        def body(i, _):
            j0 = 2 * i
            j1 = j0 + 1
            pltpu.async_copy(u.at[sidxs.at[par, j1]], rows1, g1)
            pltpu.make_async_copy(u.at[sidxs.at[par, j0]], rows0, g0).wait()
            pltpu.sync_copy(rows0, acc.at[didxs.at[par, j0]], add=True)

            @pl.when(i + 1 < G // 2)
            def _():
                pltpu.async_copy(u.at[sidxs.at[par, j0 + 2]], rows0, g0)

            pltpu.make_async_copy(u.at[sidxs.at[par, j1]], rows1, g1).wait()
            pltpu.sync_copy(rows1, acc.at[didxs.at[par, j1]], add=True)
            return 0

        lax.fori_loop(0, G // 2, body, 0)
        return 0

    lax.fori_loop(0, CPW // G, group, 0)
    plsc.subcore_barrier()
    _write_out(acc, sp, core)


_prop_call = functools.partial(
    pl.kernel,
    out_type=jax.ShapeDtypeStruct((NC, NP, D), jnp.float32),
    mesh=_mesh,
    scratch_types=[
        pltpu.VMEM((2, G, CH), jnp.int32),
        pltpu.VMEM((2, G, CH), jnp.int32),
        pltpu.VMEM((CH, D), jnp.float32),
        pltpu.VMEM((CH, D), jnp.float32),
        pltpu.VMEM((ZROWS, D), jnp.float32),
        pltpu.VMEM_SHARED((NP, D), jnp.float32),
        pltpu.SemaphoreType.DMA,
        pltpu.SemaphoreType.DMA,
        pltpu.SemaphoreType.DMA,
    ],
)(_prop_body)


BN = 2000
_GRID = N // BN


def _row_specs(n, shape=(BN, D)):
    return [pl.BlockSpec(shape, lambda i: (i, 0)) for _ in range(n)]


_PART_SPEC = pl.BlockSpec((NC, BN, D), lambda i: (0, i, 0))


def _norm_body(dp, x, normf, u0):
    d = dp[...]
    degc = d[0, :, 0:1] + d[1, :, 0:1]
    nrm = jnp.where(degc > 0, lax.rsqrt(jnp.maximum(degc, 1.0)), 0.0)
    nb = jnp.broadcast_to(nrm, (BN, D))
    normf[...] = nb
    u0[...] = x[...] * nb


def _norm_call(dp, x):
    return pl.pallas_call(
        _norm_body,
        grid=(_GRID,),
        in_specs=[_PART_SPEC] + _row_specs(1),
        out_specs=_row_specs(2),
        out_shape=[jax.ShapeDtypeStruct((N, D), jnp.float32)] * 2,
    )(dp, x)


def _mid_body(sp, nrm, h1, u1):
    spv = sp[...]
    s = (spv[0] + spv[1]) * nrm[...]
    h1[...] = s
    u1[...] = s * nrm[...]


def _mid_call(sp, nrm):
    return pl.pallas_call(
        _mid_body,
        grid=(_GRID,),
        in_specs=[_PART_SPEC] + _row_specs(1),
        out_specs=_row_specs(2),
        out_shape=[jax.ShapeDtypeStruct((N, D), jnp.float32)] * 2,
    )(sp, nrm)


def _out_body(h, h1, s2p, nrm, w_ref, b_ref, hout, uout):
    s2v = s2p[...]
    h2 = (s2v[0] + s2v[1]) * nrm[...]
    wm = w_ref[...]
    acc = jnp.dot(h[...], wm[0:D], preferred_element_type=jnp.float32,
                  precision=lax.Precision.HIGHEST)
    acc += jnp.dot(h1[...], wm[D:2 * D], preferred_element_type=jnp.float32,
                   precision=lax.Precision.HIGHEST)
    acc += jnp.dot(h2, wm[2 * D:3 * D], preferred_element_type=jnp.float32,
                   precision=lax.Precision.HIGHEST)
    o = jnp.maximum(acc + b_ref[...], 0.0)
    hout[...] = o
    uout[...] = o * nrm[...]


def _out_call(h, h1, s2p, nrm, wmat, bias):
    return pl.pallas_call(
        _out_body,
        grid=(_GRID,),
        in_specs=_row_specs(2) + [_PART_SPEC] + _row_specs(1) + [
            pl.BlockSpec((3 * D, D), lambda i: (0, 0)),
            pl.BlockSpec((1, D), lambda i: (0, 0)),
        ],
        out_specs=_row_specs(2),
        out_shape=[jax.ShapeDtypeStruct((N, D), jnp.float32)] * 2,
    )(h, h1, s2p, nrm, wmat, bias)


def kernel(x, edge_index, W1, b1, W2, b2):
    e3 = edge_index.reshape(2, NCHUNKS, CH)
    degp = _deg_call(e3)
    nrm, u0 = _norm_call(degp, x)

    s1 = _prop_call(u0, e3)
    h1, u1 = _mid_call(s1, nrm)
    s2 = _prop_call(u1, e3)
    hL1, uL1 = _out_call(x, h1, s2, nrm, W1, b1.reshape(1, D))

    s3 = _prop_call(uL1, e3)
    h21, u21 = _mid_call(s3, nrm)
    s4 = _prop_call(u21, e3)
    hL2, _ = _out_call(hL1, h21, s4, nrm, W2, b2.reshape(1, D))
    return hL2


# speedup vs baseline: 1.0545x; 1.0184x over previous
import functools

import jax
import jax.numpy as jnp
from jax import lax
from jax.experimental import pallas as pl
from jax.experimental.pallas import tpu as pltpu
from jax.experimental.pallas import tpu_sc as plsc

N = 10000
E = 320000
D = 128
NC, NS = 2, 16
NW = NC * NS
CH = 125
NCHUNKS = E // CH
CPW = NCHUNKS // NW
NP = 10240
ROWS_PER_TILE = NP // NS
ZROWS = 40
WROWS = 128
G = 16

_mesh = plsc.VectorSubcoreMesh(
    core_axis_name="c", subcore_axis_name="s", num_cores=NC, num_subcores=NS)


def _worker_id():
    return lax.axis_index("s") * NC + lax.axis_index("c")


def _zero_acc(zbuf, acc, width):
    s = lax.axis_index("s")

    def fill_zero(i, _):
        for j in range(width // 16):
            zbuf[i, pl.ds(j * 16, 16)] = jnp.zeros((16,), jnp.float32)
        return 0

    lax.fori_loop(0, ZROWS, fill_zero, 0)

    def zchunk(j, _):
        pltpu.sync_copy(zbuf, acc.at[pl.ds(s * ROWS_PER_TILE + j * ZROWS, ZROWS)])
        return 0

    lax.fori_loop(0, ROWS_PER_TILE // ZROWS, zchunk, 0)


def _write_out(acc, out, core):
    s = lax.axis_index("s")

    def wchunk(j, _):
        r0 = s * ROWS_PER_TILE + j * WROWS
        pltpu.sync_copy(acc.at[pl.ds(r0, WROWS)], out.at[core, pl.ds(r0, WROWS)])
        return 0

    lax.fori_loop(0, ROWS_PER_TILE // WROWS, wchunk, 0)


_DEG_WIN = 8


def _deg_body(e3, degp, didxs, ones_v, zbuf, acc, sem):
    core = lax.axis_index("c")
    w = _worker_id()

    def fill_ones(i, _):
        for j in range(D // 16):
            ones_v[i, pl.ds(j * 16, 16)] = jnp.ones((16,), jnp.float32)
        return 0

    lax.fori_loop(0, CH, fill_ones, 0)
    _zero_acc(zbuf, acc, D)
    pltpu.sync_copy(e3.at[1, pl.ds(w * CPW, CPW)], didxs)
    plsc.subcore_barrier()

    def body(j, _):
        pltpu.async_copy(ones_v, acc.at[didxs.at[j]], sem, add=True)

        @pl.when(j >= _DEG_WIN)
        def _():
            pltpu.make_async_copy(ones_v, acc.at[didxs.at[0]], sem).wait()

        return 0

    lax.fori_loop(0, CPW, body, 0)

    def drain(j, _):
        pltpu.make_async_copy(ones_v, acc.at[didxs.at[0]], sem).wait()
        return 0

    lax.fori_loop(0, _DEG_WIN, drain, 0)
    plsc.subcore_barrier()
    _write_out(acc, degp, core)


_deg_call = functools.partial(
    pl.kernel,
    out_type=jax.ShapeDtypeStruct((NC, NP, D), jnp.float32),
    mesh=_mesh,
    scratch_types=[
        pltpu.VMEM((CPW, CH), jnp.int32),
        pltpu.VMEM((CH, D), jnp.float32),
        pltpu.VMEM((ZROWS, D), jnp.float32),
        pltpu.VMEM_SHARED((NP, D), jnp.float32),
        pltpu.SemaphoreType.DMA,
    ],
)(_deg_body)


def _prop_body(u, e3, sp, sidxs, didxs, rows0, rows1, zbuf, acc, g0, g1, isem):
    core = lax.axis_index("c")
    w = _worker_id()

    pltpu.async_copy(e3.at[0, pl.ds(w * CPW, G)], sidxs.at[0], isem)
    pltpu.async_copy(e3.at[1, pl.ds(w * CPW, G)], didxs.at[0], isem)
    _zero_acc(zbuf, acc, D)
    plsc.subcore_barrier()

    def group(gidx, _):
        par = gidx % 2
        pltpu.make_async_copy(e3.at[0, pl.ds(0, G)], sidxs.at[par], isem).wait()
        pltpu.make_async_copy(e3.at[1, pl.ds(0, G)], didxs.at[par], isem).wait()

        @pl.when(gidx + 1 < CPW // G)
        def _():
            nrow = w * CPW + (gidx + 1) * G
            pltpu.async_copy(e3.at[0, pl.ds(nrow, G)], sidxs.at[1 - par], isem)
            pltpu.async_copy(e3.at[1, pl.ds(nrow, G)], didxs.at[1 - par], isem)

        pltpu.async_copy(u.at[sidxs.at[par, 0]], rows0, g0)

        def body(i, _):
            j0 = 2 * i
            j1 = j0 + 1
            pltpu.async_copy(u.at[sidxs.at[par, j1]], rows1, g1)
            pltpu.make_async_copy(u.at[sidxs.at[par, j0]], rows0, g0).wait()
            pltpu.sync_copy(rows0, acc.at[didxs.at[par, j0]], add=True)

            @pl.when(i + 1 < G // 2)
            def _():
                pltpu.async_copy(u.at[sidxs.at[par, j0 + 2]], rows0, g0)

            pltpu.make_async_copy(u.at[sidxs.at[par, j1]], rows1, g1).wait()
            pltpu.sync_copy(rows1, acc.at[didxs.at[par, j1]], add=True)
            return 0

        lax.fori_loop(0, G // 2, body, 0)
        return 0

    lax.fori_loop(0, CPW // G, group, 0)
    plsc.subcore_barrier()
    _write_out(acc, sp, core)


_prop_call = functools.partial(
    pl.kernel,
    out_type=jax.ShapeDtypeStruct((NC, NP, D), jnp.float32),
    mesh=_mesh,
    scratch_types=[
        pltpu.VMEM((2, G, CH), jnp.int32),
        pltpu.VMEM((2, G, CH), jnp.int32),
        pltpu.VMEM((CH, D), jnp.float32),
        pltpu.VMEM((CH, D), jnp.float32),
        pltpu.VMEM((ZROWS, D), jnp.float32),
        pltpu.VMEM_SHARED((NP, D), jnp.float32),
        pltpu.SemaphoreType.DMA,
        pltpu.SemaphoreType.DMA,
        pltpu.SemaphoreType.DMA,
    ],
)(_prop_body)


BN = 2000
_GRID = N // BN


def _row_specs(n, shape=(BN, D)):
    return [pl.BlockSpec(shape, lambda i: (i, 0)) for _ in range(n)]


_PART_SPEC = pl.BlockSpec((NC, BN, D), lambda i: (0, i, 0))


def _norm_body(dp, x, normf, u0):
    d = dp[...]
    degc = d[0, :, 0:1] + d[1, :, 0:1]
    nrm = jnp.where(degc > 0, lax.rsqrt(jnp.maximum(degc, 1.0)), 0.0)
    nb = jnp.broadcast_to(nrm, (BN, D))
    normf[...] = nb
    u0[...] = x[...] * nb


def _norm_call(dp, x):
    return pl.pallas_call(
        _norm_body,
        grid=(_GRID,),
        in_specs=[_PART_SPEC] + _row_specs(1),
        out_specs=_row_specs(2),
        out_shape=[jax.ShapeDtypeStruct((N, D), jnp.float32)] * 2,
    )(dp, x)


def _mid_body(sp, nrm, h1, u1):
    spv = sp[...]
    s = (spv[0] + spv[1]) * nrm[...]
    h1[...] = s
    u1[...] = s * nrm[...]


def _mid_call(sp, nrm):
    return pl.pallas_call(
        _mid_body,
        grid=(_GRID,),
        in_specs=[_PART_SPEC] + _row_specs(1),
        out_specs=_row_specs(2),
        out_shape=[jax.ShapeDtypeStruct((N, D), jnp.float32)] * 2,
    )(sp, nrm)


def _out_body(h, h1, s2p, nrm, w_ref, b_ref, hout, uout):
    s2v = s2p[...]
    h2 = (s2v[0] + s2v[1]) * nrm[...]
    wm = w_ref[...]
    acc = jnp.dot(h[...], wm[0:D], preferred_element_type=jnp.float32)
    acc += jnp.dot(h1[...], wm[D:2 * D], preferred_element_type=jnp.float32)
    acc += jnp.dot(h2, wm[2 * D:3 * D], preferred_element_type=jnp.float32)
    o = jnp.maximum(acc + b_ref[...], 0.0)
    hout[...] = o
    uout[...] = o * nrm[...]


def _out_call(h, h1, s2p, nrm, wmat, bias):
    return pl.pallas_call(
        _out_body,
        grid=(_GRID,),
        in_specs=_row_specs(2) + [_PART_SPEC] + _row_specs(1) + [
            pl.BlockSpec((3 * D, D), lambda i: (0, 0)),
            pl.BlockSpec((1, D), lambda i: (0, 0)),
        ],
        out_specs=_row_specs(2),
        out_shape=[jax.ShapeDtypeStruct((N, D), jnp.float32)] * 2,
    )(h, h1, s2p, nrm, wmat, bias)


def kernel(x, edge_index, W1, b1, W2, b2):
    e3 = edge_index.reshape(2, NCHUNKS, CH)
    degp = _deg_call(e3)
    nrm, u0 = _norm_call(degp, x)

    s1 = _prop_call(u0, e3)
    h1, u1 = _mid_call(s1, nrm)
    s2 = _prop_call(u1, e3)
    hL1, uL1 = _out_call(x, h1, s2, nrm, W1, b1.reshape(1, D))

    s3 = _prop_call(uL1, e3)
    h21, u21 = _mid_call(s3, nrm)
    s4 = _prop_call(u21, e3)
    hL2, _ = _out_call(hL1, h21, s4, nrm, W2, b2.reshape(1, D))
    return hL2
